# Initial kernel scaffold; baseline (speedup 1.0000x reference)
import jax
import jax.numpy as jnp
from jax.experimental import pallas as pl


def kernel(queries, keys):
    raise NotImplementedError("write your pallas kernel here")


# speedup vs baseline: 1.0172x; 1.0172x over previous
import functools

import jax
import jax.numpy as jnp
from jax import lax
from jax.experimental import pallas as pl

Q = 64
D = 128
N_KEYS = 1000000
BLK = 16384
GRID = 62
K_PAD = BLK * GRID
CHUNK = 128
N_CHUNKS = K_PAD // CHUNK
TOPK = 100
EPS = 1e-8


def _phase_a_body(q_ref, k_ref, sim_ref, m_ref):
    b = pl.program_id(0)
    q = q_ref[...]
    qn = q / (jnp.sqrt(jnp.sum(q * q, axis=-1, keepdims=True)) + EPS)
    k = k_ref[...]
    kn = k / (jnp.sqrt(jnp.sum(k * k, axis=-1, keepdims=True)) + EPS)
    sim = lax.dot_general(qn, kn, (((1,), (1,)), ((), ())),
                          preferred_element_type=jnp.float32)
    gid = b * BLK + lax.broadcasted_iota(jnp.int32, (Q, BLK), 1)
    sim = jnp.where(gid < N_KEYS, sim, -jnp.inf)
    sim_ref[...] = sim
    m_ref[...] = jnp.max(sim.reshape(Q, 8, 16, CHUNK), axis=-1)


def _phase_a(queries, keys):
    return pl.pallas_call(
        _phase_a_body,
        grid=(GRID,),
        in_specs=[
            pl.BlockSpec((Q, D), lambda b: (0, 0)),
            pl.BlockSpec((BLK, D), lambda b: (b, 0)),
        ],
        out_specs=[
            pl.BlockSpec((Q, BLK), lambda b: (0, b)),
            pl.BlockSpec((Q, 8, 16), lambda b: (0, b, 0)),
        ],
        out_shape=[
            jax.ShapeDtypeStruct((Q, K_PAD), jnp.float32),
            jax.ShapeDtypeStruct((Q, GRID * 8, 16), jnp.float32),
        ],
    )(queries, keys)


def kernel(queries, keys):
    sim, m = _phase_a(queries, keys)
    vals, idx = lax.top_k(sim[:, :N_KEYS], TOPK)
    return vals, idx


# speedup vs baseline: 17.6963x; 17.3978x over previous
import functools

import jax
import jax.numpy as jnp
from jax import lax
from jax.experimental import pallas as pl
from jax.experimental.pallas import tpu as pltpu
from jax.experimental.pallas import tpu_sc as plsc

Q = 64
D = 128
N_KEYS = 1000000
BLK = 16384
GRID = 62
K_PAD = BLK * GRID
CHUNK = 128
N_CHUNKS = K_PAD // CHUNK
NV_M = N_CHUNKS // 16
ROWS16 = K_PAD // 16
TOPK = 100
CAND = 128
EPS = 1e-8
NEG = float("-inf")


def _phase_a_body(q_ref, k_ref, sim_ref, m_ref):
    b = pl.program_id(0)
    q = q_ref[...]
    qn = q / (jnp.sqrt(jnp.sum(q * q, axis=-1, keepdims=True)) + EPS)
    k = k_ref[...]
    kn = k / (jnp.sqrt(jnp.sum(k * k, axis=-1, keepdims=True)) + EPS)
    sim = lax.dot_general(qn, kn, (((1,), (1,)), ((), ())),
                          preferred_element_type=jnp.float32)
    gid = b * BLK + lax.broadcasted_iota(jnp.int32, (Q, BLK), 1)
    sim = jnp.where(gid < N_KEYS, sim, NEG)
    sim_ref[...] = sim
    m_ref[...] = jnp.max(sim.reshape(Q, 8, 16, CHUNK), axis=-1)


def _phase_a(queries, keys):
    return pl.pallas_call(
        _phase_a_body,
        grid=(GRID,),
        in_specs=[
            pl.BlockSpec((Q, D), lambda b: (0, 0)),
            pl.BlockSpec((BLK, D), lambda b: (b, 0)),
        ],
        out_specs=[
            pl.BlockSpec((Q, BLK), lambda b: (0, b)),
            pl.BlockSpec((Q, 8, 16), lambda b: (0, b, 0)),
        ],
        out_shape=[
            jax.ShapeDtypeStruct((Q, K_PAD), jnp.float32),
            jax.ShapeDtypeStruct((Q, NV_M, 16), jnp.float32),
        ],
    )(queries, keys)


_IOTA = None


def _scal(x):
    return x[0] if x.ndim else x


_GDN = lax.GatherDimensionNumbers(offset_dims=(), collapsed_slice_dims=(0,),
                                  start_index_map=(0,))


def _take16(v, idx):
    return lax.gather(v, idx[:, None], _GDN, (1,),
                      mode=lax.GatherScatterMode.PROMISE_IN_BOUNDS)


def _vmax16(v, iota):
    for s in (8, 4, 2, 1):
        v = jnp.maximum(v, _take16(v, iota ^ s))
    return v[0]


def _prefix_sum16(v, iota):
    for s in (1, 2, 4, 8):
        w = jnp.where(iota >= s, _take16(v, (iota - s) & 15), 0)
        v = v + w
    return v


def _vmin16i(v, iota):
    for s in (8, 4, 2, 1):
        v = jnp.minimum(v, _take16(v, iota ^ s))
    return v[0]


def _ffs_i(mask, iota):
    v = jnp.where(mask, iota, 16)
    for s in (8, 4, 2, 1):
        v = jnp.minimum(v, _take16(v, iota ^ s))
    return v[0]


def _popcnt_i(mask, iota):
    v = jnp.where(mask, 1, 0)
    for s in (8, 4, 2, 1):
        v = v + _take16(v, iota ^ s)
    return v[0]


def _sc_topk(m, sim3):
    mesh = plsc.VectorSubcoreMesh(core_axis_name="c", subcore_axis_name="s")
    info = plsc.get_sparse_core_info()
    nc = info.num_cores

    @functools.partial(
        pl.kernel,
        mesh=mesh,
        out_type=[
            jax.ShapeDtypeStruct((Q, 8, 16), jnp.float32),
            jax.ShapeDtypeStruct((Q, 8, 16), jnp.int32),
        ],
        scratch_types=[
            pltpu.VMEM((NV_M, 16), jnp.float32),
            pltpu.VMEM((31, 16), jnp.float32),
            pltpu.VMEM((2, 16), jnp.float32),
            pltpu.VMEM((8, 16), jnp.int32),
            pltpu.VMEM((8, 16), jnp.int32),
            pltpu.VMEM((128,), jnp.int32),
            pltpu.VMEM((128, 128), jnp.float32),
            pltpu.VMEM((64, 16), jnp.float32),
            pltpu.VMEM((4, 16), jnp.float32),
            pltpu.VMEM((8, 16), jnp.float32),
            pltpu.VMEM((8, 16), jnp.int32),
            pltpu.SemaphoreType.DMA,
        ],
    )
    def sc_kernel(m_hbm, sim3_hbm, vout_hbm, iout_hbm,
                  m_v, pm_v, pms_v, cidtmp_v, cid_v, idx_v, cand_v,
                  cp_v, cps_v, vals_v, ids_v, sem):
        wid = lax.axis_index("s") * nc + lax.axis_index("c")
        iota = lax.iota(jnp.int32, 16)

        def per_query(qi, _):
            q = wid * 2 + qi
            pltpu.sync_copy(m_hbm.at[q], m_v)

            def build_pm(g, _):
                acc = jnp.full((16,), NEG, jnp.float32)
                for j in range(16):
                    sm = _vmax16(m_v[g * 16 + j], iota)
                    acc = jnp.where(iota == j, sm, acc)
                pm_v[g] = acc
                return 0
            lax.fori_loop(0, 31, build_pm, 0)

            def build_pms(g, _):
                sm = _vmax16(pm_v[g], iota)
                h = g >> 4
                pms_v[h] = jnp.where(iota == (g & 15), sm, pms_v[h])
                return 0
            pms_v[0] = jnp.full((16,), NEG, jnp.float32)
            pms_v[1] = jnp.full((16,), NEG, jnp.float32)
            lax.fori_loop(0, 31, build_pms, 0)

            def extract_chunk(i, _):
                t = _vmax16(jnp.maximum(pms_v[0], pms_v[1]), iota)
                f0 = _ffs_i(pms_v[0] == t, iota)
                f1 = _ffs_i(pms_v[1] == t, iota)
                g = jnp.where(f0 < 16, f0, 16 + f1)
                a = pm_v[g]
                b = _ffs_i(a == t, iota)
                r = g * 16 + b
                mv = m_v[r]
                l = _ffs_i(mv == t, iota)
                cidx = r * 16 + l
                row = i >> 4
                cidtmp_v[row] = jnp.where(iota == (i & 15), cidx, cidtmp_v[row])
                upd = jnp.where(iota == l, NEG, mv)
                m_v[r] = upd
                nm = _vmax16(upd, iota)
                pa = jnp.where(iota == b, nm, a)
                pm_v[g] = pa
                nm2 = _vmax16(pa, iota)
                h = g >> 4
                pms_v[h] = jnp.where(iota == (g & 15), nm2, pms_v[h])
                return 0
            lax.fori_loop(0, TOPK, extract_chunk, 0)

            pad6 = 7813 + ((96 + iota) & 63)
            cidtmp_v[6] = jnp.where(iota >= TOPK - 96, pad6, cidtmp_v[6])
            cidtmp_v[7] = 7813 + ((112 + iota) & 63)

            BIGI = jnp.int32(2 ** 30)
            rmin = jnp.full((16,), BIGI, jnp.int32)
            for r in range(8):
                rmin = jnp.where(iota == r, _vmin16i(cidtmp_v[r], iota), rmin)

            def sort_step(j, rmin):
                t = _vmin16i(rmin, iota)
                r = _ffs_i(rmin == t, iota)
                rowv = cidtmp_v[r]
                l = _ffs_i(rowv == t, iota)
                jr = j >> 4
                cid_v[jr] = jnp.where(iota == (j & 15), t, cid_v[jr])
                upd = jnp.where(iota == l, BIGI, rowv)
                cidtmp_v[r] = upd
                nm = _vmin16i(upd, iota)
                return jnp.where(iota == r, nm, rmin)
            lax.fori_loop(0, 128, sort_step, rmin)

            qbase = q * N_CHUNKS
            for jr in range(8):
                idx_v[pl.ds(jr * 16, 16)] = qbase + cid_v[jr]
            pltpu.async_copy(sim3_hbm.at[idx_v], cand_v, sem).wait()

            def build_cp(a, _):
                acc = jnp.full((16,), NEG, jnp.float32)
                for j in range(16):
                    cvj = cand_v[2 * a + (j >> 3), pl.ds((j & 7) * 16, 16)]
                    sm = _vmax16(cvj, iota)
                    acc = jnp.where(iota == j, sm, acc)
                cp_v[a] = acc
                return 0
            lax.fori_loop(0, 64, build_cp, 0)

            def build_cps(a, _):
                sm = _vmax16(cp_v[a], iota)
                h = a >> 4
                cps_v[h] = jnp.where(iota == (a & 15), sm, cps_v[h])
                return 0
            for h in range(4):
                cps_v[h] = jnp.full((16,), NEG, jnp.float32)
            lax.fori_loop(0, 64, build_cps, 0)

            def extract_cand(i, _):
                t = _vmax16(jnp.maximum(jnp.maximum(cps_v[0], cps_v[1]),
                                        jnp.maximum(cps_v[2], cps_v[3])), iota)
                f0 = _ffs_i(cps_v[0] == t, iota)
                f1 = _ffs_i(cps_v[1] == t, iota)
                f2 = _ffs_i(cps_v[2] == t, iota)
                f3 = _ffs_i(cps_v[3] == t, iota)
                a = jnp.where(f0 < 16, f0,
                    jnp.where(f1 < 16, 16 + f1,
                    jnp.where(f2 < 16, 32 + f2, 48 + f3)))
                av = cp_v[a]
                b = _ffs_i(av == t, iota)
                slot = 2 * a + (b >> 3)
                sub = b & 7
                cv = cand_v[slot, pl.ds(sub * 16, 16)]
                l = _ffs_i(cv == t, iota)
                BIGI = jnp.int32(2 ** 30)
                cid = _vmin16i(jnp.where(iota == (slot & 15),
                                         cid_v[slot >> 4], BIGI), iota)
                gidx = cid * CHUNK + sub * 16 + l
                irow = i >> 4
                vals_v[irow] = jnp.where(iota == (i & 15), t, vals_v[irow])
                ids_v[irow] = jnp.where(iota == (i & 15), gidx, ids_v[irow])
                upd = jnp.where(iota == l, NEG, cv)
                cand_v[slot, pl.ds(sub * 16, 16)] = upd
                nm = _vmax16(upd, iota)
                pa = jnp.where(iota == b, nm, av)
                cp_v[a] = pa
                nm2 = _vmax16(pa, iota)
                h = a >> 4
                cps_v[h] = jnp.where(iota == (a & 15), nm2, cps_v[h])
                return 0
            lax.fori_loop(0, TOPK, extract_cand, 0)

            pltpu.sync_copy(vals_v, vout_hbm.at[q])
            pltpu.sync_copy(ids_v, iout_hbm.at[q])
            return 0

        lax.fori_loop(0, 2, per_query, 0)

    return sc_kernel(m, sim3)


def kernel(queries, keys):
    sim, m = _phase_a(queries, keys)
    sim3 = sim.reshape(Q * N_CHUNKS, CHUNK)
    vals, idx = _sc_topk(m, sim3)
    return (vals.reshape(Q, 128)[:, :TOPK], idx.reshape(Q, 128)[:, :TOPK])


# speedup vs baseline: 17.7749x; 1.0044x over previous
import functools

import jax
import jax.numpy as jnp
from jax import lax
from jax.experimental import pallas as pl
from jax.experimental.pallas import tpu as pltpu
from jax.experimental.pallas import tpu_sc as plsc

Q = 64
D = 128
N_KEYS = 1000000
BLK = 32768
GRID = 31
K_PAD = BLK * GRID
CHUNK = 128
N_CHUNKS = K_PAD // CHUNK
NV_M = N_CHUNKS // 16
ROWS16 = K_PAD // 16
TOPK = 100
CAND = 128
EPS = 1e-8
NEG = float("-inf")


def _phase_a_body(q_ref, k_ref, sim_ref, m_ref):
    b = pl.program_id(0)
    q = q_ref[...]
    qn = q / (jnp.sqrt(jnp.sum(q * q, axis=-1, keepdims=True)) + EPS)
    k = k_ref[...]
    kn = k / (jnp.sqrt(jnp.sum(k * k, axis=-1, keepdims=True)) + EPS)
    sim = lax.dot_general(qn, kn, (((1,), (1,)), ((), ())),
                          preferred_element_type=jnp.float32)
    gid = b * BLK + lax.broadcasted_iota(jnp.int32, (Q, BLK), 1)
    sim = jnp.where(gid < N_KEYS, sim, NEG)
    sim_ref[...] = sim
    m_ref[...] = jnp.max(sim.reshape(Q, BLK // 2048, 16, CHUNK), axis=-1)


def _phase_a(queries, keys):
    return pl.pallas_call(
        _phase_a_body,
        grid=(GRID,),
        in_specs=[
            pl.BlockSpec((Q, D), lambda b: (0, 0)),
            pl.BlockSpec((BLK, D), lambda b: (b, 0)),
        ],
        out_specs=[
            pl.BlockSpec((Q, BLK), lambda b: (0, b)),
            pl.BlockSpec((Q, BLK // 2048, 16), lambda b: (0, b, 0)),
        ],
        out_shape=[
            jax.ShapeDtypeStruct((Q, K_PAD), jnp.float32),
            jax.ShapeDtypeStruct((Q, NV_M, 16), jnp.float32),
        ],
    )(queries, keys)


_IOTA = None


def _scal(x):
    return x[0] if x.ndim else x


_GDN = lax.GatherDimensionNumbers(offset_dims=(), collapsed_slice_dims=(0,),
                                  start_index_map=(0,))


def _take16(v, idx):
    return lax.gather(v, idx[:, None], _GDN, (1,),
                      mode=lax.GatherScatterMode.PROMISE_IN_BOUNDS)


def _vmax16(v, iota):
    for s in (8, 4, 2, 1):
        v = jnp.maximum(v, _take16(v, iota ^ s))
    return v[0]


def _prefix_sum16(v, iota):
    for s in (1, 2, 4, 8):
        w = jnp.where(iota >= s, _take16(v, (iota - s) & 15), 0)
        v = v + w
    return v


def _vmin16i(v, iota):
    for s in (8, 4, 2, 1):
        v = jnp.minimum(v, _take16(v, iota ^ s))
    return v[0]


def _ffs_i(mask, iota):
    v = jnp.where(mask, iota, 16)
    for s in (8, 4, 2, 1):
        v = jnp.minimum(v, _take16(v, iota ^ s))
    return v[0]


def _popcnt_i(mask, iota):
    v = jnp.where(mask, 1, 0)
    for s in (8, 4, 2, 1):
        v = v + _take16(v, iota ^ s)
    return v[0]


def _sc_topk(m, sim3):
    mesh = plsc.VectorSubcoreMesh(core_axis_name="c", subcore_axis_name="s")
    info = plsc.get_sparse_core_info()
    nc = info.num_cores

    @functools.partial(
        pl.kernel,
        mesh=mesh,
        out_type=[
            jax.ShapeDtypeStruct((Q, 8, 16), jnp.float32),
            jax.ShapeDtypeStruct((Q, 8, 16), jnp.int32),
        ],
        scratch_types=[
            pltpu.VMEM((NV_M, 16), jnp.float32),
            pltpu.VMEM((31, 16), jnp.float32),
            pltpu.VMEM((2, 16), jnp.float32),
            pltpu.VMEM((8, 16), jnp.int32),
            pltpu.VMEM((8, 16), jnp.int32),
            pltpu.VMEM((128,), jnp.int32),
            pltpu.VMEM((128, 128), jnp.float32),
            pltpu.VMEM((64, 16), jnp.float32),
            pltpu.VMEM((4, 16), jnp.float32),
            pltpu.VMEM((8, 16), jnp.float32),
            pltpu.VMEM((8, 16), jnp.int32),
            pltpu.SemaphoreType.DMA,
        ],
    )
    def sc_kernel(m_hbm, sim3_hbm, vout_hbm, iout_hbm,
                  m_v, pm_v, pms_v, cidtmp_v, cid_v, idx_v, cand_v,
                  cp_v, cps_v, vals_v, ids_v, sem):
        wid = lax.axis_index("s") * nc + lax.axis_index("c")
        iota = lax.iota(jnp.int32, 16)

        def per_query(qi, _):
            q = wid * 2 + qi
            pltpu.sync_copy(m_hbm.at[q], m_v)

            def build_pm(g, _):
                acc = jnp.full((16,), NEG, jnp.float32)
                for j in range(16):
                    sm = _vmax16(m_v[g * 16 + j], iota)
                    acc = jnp.where(iota == j, sm, acc)
                pm_v[g] = acc
                return 0
            lax.fori_loop(0, 31, build_pm, 0)

            def build_pms(g, _):
                sm = _vmax16(pm_v[g], iota)
                h = g >> 4
                pms_v[h] = jnp.where(iota == (g & 15), sm, pms_v[h])
                return 0
            pms_v[0] = jnp.full((16,), NEG, jnp.float32)
            pms_v[1] = jnp.full((16,), NEG, jnp.float32)
            lax.fori_loop(0, 31, build_pms, 0)

            def extract_chunk(i, _):
                t = _vmax16(jnp.maximum(pms_v[0], pms_v[1]), iota)
                f0 = _ffs_i(pms_v[0] == t, iota)
                f1 = _ffs_i(pms_v[1] == t, iota)
                g = jnp.where(f0 < 16, f0, 16 + f1)
                a = pm_v[g]
                b = _ffs_i(a == t, iota)
                r = g * 16 + b
                mv = m_v[r]
                l = _ffs_i(mv == t, iota)
                cidx = r * 16 + l
                row = i >> 4
                cidtmp_v[row] = jnp.where(iota == (i & 15), cidx, cidtmp_v[row])
                upd = jnp.where(iota == l, NEG, mv)
                m_v[r] = upd
                nm = _vmax16(upd, iota)
                pa = jnp.where(iota == b, nm, a)
                pm_v[g] = pa
                nm2 = _vmax16(pa, iota)
                h = g >> 4
                pms_v[h] = jnp.where(iota == (g & 15), nm2, pms_v[h])
                return 0
            lax.fori_loop(0, TOPK, extract_chunk, 0)

            pad6 = 7813 + ((96 + iota) & 63)
            cidtmp_v[6] = jnp.where(iota >= TOPK - 96, pad6, cidtmp_v[6])
            cidtmp_v[7] = 7813 + ((112 + iota) & 63)

            BIGI = jnp.int32(2 ** 30)
            rmin = jnp.full((16,), BIGI, jnp.int32)
            for r in range(8):
                rmin = jnp.where(iota == r, _vmin16i(cidtmp_v[r], iota), rmin)

            def sort_step(j, rmin):
                t = _vmin16i(rmin, iota)
                r = _ffs_i(rmin == t, iota)
                rowv = cidtmp_v[r]
                l = _ffs_i(rowv == t, iota)
                jr = j >> 4
                cid_v[jr] = jnp.where(iota == (j & 15), t, cid_v[jr])
                upd = jnp.where(iota == l, BIGI, rowv)
                cidtmp_v[r] = upd
                nm = _vmin16i(upd, iota)
                return jnp.where(iota == r, nm, rmin)
            lax.fori_loop(0, 128, sort_step, rmin)

            qbase = q * N_CHUNKS
            for jr in range(8):
                idx_v[pl.ds(jr * 16, 16)] = qbase + cid_v[jr]
            pltpu.async_copy(sim3_hbm.at[idx_v], cand_v, sem).wait()

            def build_cp(a, _):
                acc = jnp.full((16,), NEG, jnp.float32)
                for j in range(16):
                    cvj = cand_v[2 * a + (j >> 3), pl.ds((j & 7) * 16, 16)]
                    sm = _vmax16(cvj, iota)
                    acc = jnp.where(iota == j, sm, acc)
                cp_v[a] = acc
                return 0
            lax.fori_loop(0, 64, build_cp, 0)

            def build_cps(a, _):
                sm = _vmax16(cp_v[a], iota)
                h = a >> 4
                cps_v[h] = jnp.where(iota == (a & 15), sm, cps_v[h])
                return 0
            for h in range(4):
                cps_v[h] = jnp.full((16,), NEG, jnp.float32)
            lax.fori_loop(0, 64, build_cps, 0)

            def extract_cand(i, _):
                t = _vmax16(jnp.maximum(jnp.maximum(cps_v[0], cps_v[1]),
                                        jnp.maximum(cps_v[2], cps_v[3])), iota)
                f0 = _ffs_i(cps_v[0] == t, iota)
                f1 = _ffs_i(cps_v[1] == t, iota)
                f2 = _ffs_i(cps_v[2] == t, iota)
                f3 = _ffs_i(cps_v[3] == t, iota)
                a = jnp.where(f0 < 16, f0,
                    jnp.where(f1 < 16, 16 + f1,
                    jnp.where(f2 < 16, 32 + f2, 48 + f3)))
                av = cp_v[a]
                b = _ffs_i(av == t, iota)
                slot = 2 * a + (b >> 3)
                sub = b & 7
                cv = cand_v[slot, pl.ds(sub * 16, 16)]
                l = _ffs_i(cv == t, iota)
                BIGI = jnp.int32(2 ** 30)
                cid = _vmin16i(jnp.where(iota == (slot & 15),
                                         cid_v[slot >> 4], BIGI), iota)
                gidx = cid * CHUNK + sub * 16 + l
                irow = i >> 4
                vals_v[irow] = jnp.where(iota == (i & 15), t, vals_v[irow])
                ids_v[irow] = jnp.where(iota == (i & 15), gidx, ids_v[irow])
                upd = jnp.where(iota == l, NEG, cv)
                cand_v[slot, pl.ds(sub * 16, 16)] = upd
                nm = _vmax16(upd, iota)
                pa = jnp.where(iota == b, nm, av)
                cp_v[a] = pa
                nm2 = _vmax16(pa, iota)
                h = a >> 4
                cps_v[h] = jnp.where(iota == (a & 15), nm2, cps_v[h])
                return 0
            lax.fori_loop(0, TOPK, extract_cand, 0)

            pltpu.sync_copy(vals_v, vout_hbm.at[q])
            pltpu.sync_copy(ids_v, iout_hbm.at[q])
            return 0

        lax.fori_loop(0, 2, per_query, 0)

    return sc_kernel(m, sim3)


def kernel(queries, keys):
    sim, m = _phase_a(queries, keys)
    sim3 = sim.reshape(Q * N_CHUNKS, CHUNK)
    vals, idx = _sc_topk(m, sim3)
    return (vals.reshape(Q, 128)[:, :TOPK], idx.reshape(Q, 128)[:, :TOPK])
